# baseline (speedup 1.0000x reference)
import functools

import jax
import jax.numpy as jnp
from jax import lax
from jax.experimental import pallas as pl
from jax.experimental.pallas import tpu as pltpu
from jax.experimental.pallas import tpu_sc as plsc

_LANES = 16
_NUM_WORKERS = 32


@functools.cache
def _build(B, F, VOCAB, FD):
    bpw = B // _NUM_WORKERS
    assert B % (_NUM_WORKERS * _LANES) == 0
    n_chunks = bpw // _LANES

    mesh = plsc.VectorSubcoreMesh(core_axis_name="c", subcore_axis_name="s")

    @functools.partial(
        pl.kernel,
        mesh=mesh,
        out_type=jax.ShapeDtypeStruct((B,), jnp.float32),
        scratch_types=[
            pltpu.VMEM((F, bpw), jnp.int32),
            pltpu.VMEM((F, bpw), jnp.float32),
            pltpu.VMEM((FD, bpw), jnp.float32),
            pltpu.VMEM((FD, _LANES), jnp.float32),
            pltpu.VMEM((bpw,), jnp.float32),
            pltpu.SemaphoreType.DMA,
        ],
    )
    def k(idx_hbm, tflat_hbm, xd_hbm, w_hbm, out_hbm,
          idx_v, val_v, xd_v, w_v, acc_v, sem):
        wid = lax.axis_index("s") * 2 + lax.axis_index("c")
        base = wid * bpw

        pltpu.sync_copy(idx_hbm.at[:, pl.ds(base, bpw)], idx_v)
        pltpu.sync_copy(xd_hbm.at[:, pl.ds(base, bpw)], xd_v)
        pltpu.sync_copy(w_hbm, w_v)

        for f in range(1, F):
            off = jnp.full((_LANES,), f * VOCAB, jnp.int32)
            for c in range(n_chunks):
                sl = (f, pl.ds(c * _LANES, _LANES))
                idx_v[sl] = idx_v[sl] + off

        copies = [
            pltpu.async_copy(tflat_hbm.at[idx_v.at[f]], val_v.at[f], sem)
            for f in range(F)
        ]
        for cp in copies:
            cp.wait()

        w_splat = [w_v[d, :] for d in range(FD)]

        for c in range(n_chunks):
            sl = pl.ds(c * _LANES, _LANES)
            s = val_v[0, sl]
            for f in range(1, F):
                s = s + val_v[f, sl]
            for d in range(FD):
                s = s + w_splat[d] * xd_v[d, sl]
            acc_v[sl] = s

        pltpu.sync_copy(acc_v, out_hbm.at[pl.ds(base, bpw)])

    return k


def kernel(x_sparse, x_dense, table, W_dense):
    F, VOCAB = table.shape
    B, FD = x_dense.shape
    idx_t = x_sparse.T.astype(jnp.int32)
    tflat = table.reshape(-1)
    xd_t = x_dense.T
    w = jnp.broadcast_to(W_dense.reshape(FD, 1), (FD, _LANES))
    out = _build(B, F, VOCAB, FD)(idx_t, tflat, xd_t, w)
    return out.reshape(B, 1)
